# baseline (speedup 1.0000x reference)
import functools

import jax
import jax.numpy as jnp
from jax import lax
from jax.experimental import pallas as pl
from jax.experimental.pallas import tpu as pltpu
from jax.experimental.pallas import tpu_sc as plsc

_BATCH = 4096
_MAX_LEN = 200
_EMBED = 32
_VOCAB = 1000000
_B = _BATCH * _MAX_LEN
_NC = 2
_NS = 16
_NW = _NC * _NS

_BPW = _B // _NW
_CHUNK = _MAX_LEN
_NCHUNK = _BPW // _CHUNK

_TBLK = 32
_TGRID = (_VOCAB + _TBLK * 128 - 1) // (_TBLK * 128)
_OUTR = _TBLK * 128 * _EMBED // 128
_VPAD = _TGRID * _TBLK * 128


@jax.jit
def _relayout_tc(table_t):
    def k(tt_ref, out_ref):
        x = tt_ref[...]
        y = x.T
        y3 = y.reshape(_OUTR, 4, _EMBED)
        out_ref[...] = jnp.concatenate(
            [y3[:, g, :] for g in range(4)], axis=1)

    return pl.pallas_call(
        k,
        grid=(_TGRID,),
        in_specs=[pl.BlockSpec((_EMBED, _TBLK * 128), lambda i: (0, i))],
        out_specs=pl.BlockSpec((_OUTR, 128), lambda i: (i, 0)),
        out_shape=jax.ShapeDtypeStruct((_TGRID * _OUTR, 128), jnp.float32),
    )(table_t)


@jax.jit
def _embedding_sc(idx_flat, table):
    mesh = plsc.VectorSubcoreMesh(core_axis_name="c", subcore_axis_name="s")

    @functools.partial(
        pl.kernel,
        mesh=mesh,
        out_type=jax.ShapeDtypeStruct((_BATCH, _MAX_LEN, _EMBED), jnp.float32),
        scratch_types=[
            pltpu.VMEM((_BPW,), jnp.int32),
            pltpu.VMEM((2, _CHUNK, _EMBED), jnp.float32),
            pltpu.SemaphoreType.DMA((2,)),
            pltpu.SemaphoreType.DMA((2,)),
        ],
        compiler_params=pltpu.CompilerParams(use_tc_tiling_on_sc=False),
    )
    def k(idx_hbm, table_hbm, out_hbm, idx_v, rows_v, gsem, wsem):
        wid = lax.axis_index("s") * _NC + lax.axis_index("c")
        base = wid * _BPW
        pltpu.sync_copy(idx_hbm.at[pl.ds(base, _BPW)], idx_v)

        def g_desc(i, b):
            return pltpu.make_async_copy(
                table_hbm.at[idx_v.at[pl.ds(i * _CHUNK, _CHUNK)]],
                rows_v.at[b], gsem.at[b])

        row0 = wid * _NCHUNK

        def w_desc(i, b):
            return pltpu.make_async_copy(
                rows_v.at[b], out_hbm.at[row0 + i], wsem.at[b])

        g_desc(0, 0).start()
        for i in range(_NCHUNK):
            b = i % 2
            nb = (i + 1) % 2
            if i + 1 < _NCHUNK:
                if i >= 1:
                    w_desc(i - 1, nb).wait()
                g_desc(i + 1, nb).start()
            g_desc(i, b).wait()
            w_desc(i, b).start()
        w_desc(_NCHUNK - 2, (_NCHUNK - 2) % 2).wait()
        w_desc(_NCHUNK - 1, (_NCHUNK - 1) % 2).wait()

    return k(idx_flat, table)


def kernel(inputs, table):
    idx_flat = inputs.reshape(-1).astype(jnp.int32)
    tlin = _relayout_tc(table.T)
    return _embedding_sc(idx_flat, tlin.reshape(_VPAD, _EMBED))


# speedup vs baseline: 1.3170x; 1.3170x over previous
import functools

import jax
import jax.numpy as jnp
from jax import lax
from jax.experimental import pallas as pl
from jax.experimental.pallas import tpu as pltpu
from jax.experimental.pallas import tpu_sc as plsc

_BATCH = 4096
_MAX_LEN = 200
_EMBED = 32
_VOCAB = 1000000
_B = _BATCH * _MAX_LEN
_NC = 2
_NS = 16
_NW = _NC * _NS

_BPW = _B // _NW
_CHUNK = 1600
_NCHUNK = _BPW // _CHUNK

_TBLK = 32
_TGRID = (_VOCAB + _TBLK * 128 - 1) // (_TBLK * 128)
_OUTR = _TBLK * 128 * _EMBED // 128
_VPAD = _TGRID * _TBLK * 128


@jax.jit
def _relayout_tc(table_t):
    def k(tt_ref, out_ref):
        for g in range(_TBLK // 4):
            xg = tt_ref[:, g * 512:(g + 1) * 512]
            z = jnp.concatenate(
                [xg[:, a * 128:(a + 1) * 128] for a in range(4)], axis=0)
            out_ref[g * 128:(g + 1) * 128, :] = z.T

    return pl.pallas_call(
        k,
        grid=(_TGRID,),
        in_specs=[pl.BlockSpec((_EMBED, _TBLK * 128), lambda i: (0, i))],
        out_specs=pl.BlockSpec((_OUTR, 128), lambda i: (i, 0)),
        out_shape=jax.ShapeDtypeStruct((_TGRID * _OUTR, 128), jnp.float32),
    )(table_t)


@jax.jit
def _embedding_sc(idx_flat, table):
    mesh = plsc.VectorSubcoreMesh(core_axis_name="c", subcore_axis_name="s")

    @functools.partial(
        pl.kernel,
        mesh=mesh,
        out_type=jax.ShapeDtypeStruct((_B, _EMBED), jnp.float32),
        scratch_types=[
            pltpu.VMEM((_BPW,), jnp.int32),
            pltpu.VMEM((2, _CHUNK, _EMBED), jnp.float32),
            pltpu.SemaphoreType.DMA((2,)),
            pltpu.SemaphoreType.DMA((2,)),
        ],
        compiler_params=pltpu.CompilerParams(use_tc_tiling_on_sc=False),
    )
    def k(idx_hbm, table_hbm, out_hbm, idx_v, rows_v, gsem, wsem):
        wid = lax.axis_index("s") * _NC + lax.axis_index("c")
        base = wid * _BPW
        pltpu.sync_copy(idx_hbm.at[pl.ds(base, _BPW)], idx_v)

        def g_desc(i, b):
            return pltpu.make_async_copy(
                table_hbm.at[idx_v.at[pl.ds(i * _CHUNK, _CHUNK)]],
                rows_v.at[b], gsem.at[b])

        def w_desc(i, b):
            return pltpu.make_async_copy(
                rows_v.at[b],
                out_hbm.at[pl.ds(base + i * _CHUNK, _CHUNK)], wsem.at[b])

        g_desc(0, 0).start()
        for i in range(_NCHUNK):
            b = i % 2
            nb = (i + 1) % 2
            if i + 1 < _NCHUNK:
                if i >= 1:
                    w_desc(i - 1, nb).wait()
                g_desc(i + 1, nb).start()
            g_desc(i, b).wait()
            w_desc(i, b).start()
        w_desc(_NCHUNK - 2, (_NCHUNK - 2) % 2).wait()
        w_desc(_NCHUNK - 1, (_NCHUNK - 1) % 2).wait()

    return k(idx_flat, table)


def kernel(inputs, table):
    v = inputs.reshape(-1).astype(jnp.int32)
    idx_flat = (
        (v & ~jnp.int32(511))
        + ((v & 127) << 2)
        + ((v >> 7) & 3)
    )
    tlin = _relayout_tc(table.T)
    out = _embedding_sc(idx_flat, tlin.reshape(_VPAD, _EMBED))
    return out.reshape(_BATCH, _MAX_LEN, _EMBED)


# speedup vs baseline: 1.8806x; 1.4280x over previous
import functools

import jax
import jax.numpy as jnp
from jax import lax
from jax.experimental import pallas as pl
from jax.experimental.pallas import tpu as pltpu
from jax.experimental.pallas import tpu_sc as plsc

_BATCH = 4096
_MAX_LEN = 200
_EMBED = 32
_VOCAB = 1000000
_B = _BATCH * _MAX_LEN
_NC = 2
_NS = 16
_NW = _NC * _NS

_BPW = _B // _NW
_CHUNK = 1600
_NCHUNK = _BPW // _CHUNK

_TBLK = 32
_TGRID = (_VOCAB + _TBLK * 128 - 1) // (_TBLK * 128)
_OUTR = _TBLK * 128 * _EMBED // 128
_VPAD = _TGRID * _TBLK * 128


@jax.jit
def _relayout_tc(table_t):
    def k(tt_ref, out_ref):
        for g in range(_TBLK // 4):
            xg = tt_ref[:, g * 512:(g + 1) * 512]
            z = jnp.concatenate(
                [xg[:, a * 128:(a + 1) * 128] for a in range(4)], axis=0)
            out_ref[g * 128:(g + 1) * 128, :] = z.T

    return pl.pallas_call(
        k,
        grid=(_TGRID,),
        in_specs=[pl.BlockSpec((_EMBED, _TBLK * 128), lambda i: (0, i))],
        out_specs=pl.BlockSpec((_OUTR, 128), lambda i: (i, 0)),
        out_shape=jax.ShapeDtypeStruct((_TGRID * _OUTR, 128), jnp.float32),
    )(table_t)


@jax.jit
def _format_tc(xlin):
    def k(in_ref, out_ref):
        for kt in range(_MAX_LEN // 4):
            z = in_ref[kt * 128:(kt + 1) * 128, :]
            out_ref[4 * kt:4 * (kt + 1), :, :] = z.T.reshape(4, _EMBED, 128)

    return pl.pallas_call(
        k,
        grid=(_BATCH // 128,),
        in_specs=[pl.BlockSpec((50 * 128, 128), lambda w: (w, 0))],
        out_specs=pl.BlockSpec(
            (_MAX_LEN, _EMBED, 128), lambda w: (0, 0, w)),
        out_shape=jax.ShapeDtypeStruct(
            (_MAX_LEN, _EMBED, _BATCH), jnp.float32),
    )(xlin)


@jax.jit
def _embedding_sc(idx_flat, table):
    mesh = plsc.VectorSubcoreMesh(core_axis_name="c", subcore_axis_name="s")

    @functools.partial(
        pl.kernel,
        mesh=mesh,
        out_type=jax.ShapeDtypeStruct((_B, _EMBED), jnp.float32),
        scratch_types=[
            pltpu.VMEM((_BPW,), jnp.int32),
            pltpu.VMEM((2, _CHUNK, _EMBED), jnp.float32),
            pltpu.SemaphoreType.DMA((2,)),
            pltpu.SemaphoreType.DMA((2,)),
        ],
        compiler_params=pltpu.CompilerParams(use_tc_tiling_on_sc=False),
    )
    def k(idx_hbm, table_hbm, out_hbm, idx_v, rows_v, gsem, wsem):
        wid = lax.axis_index("s") * _NC + lax.axis_index("c")
        base = wid * _BPW
        pltpu.sync_copy(idx_hbm.at[pl.ds(base, _BPW)], idx_v)

        def g_desc(i, b):
            return pltpu.make_async_copy(
                table_hbm.at[idx_v.at[pl.ds(i * _CHUNK, _CHUNK)]],
                rows_v.at[b], gsem.at[b])

        def w_desc(i, b):
            return pltpu.make_async_copy(
                rows_v.at[b],
                out_hbm.at[pl.ds(base + i * _CHUNK, _CHUNK)], wsem.at[b])

        g_desc(0, 0).start()
        for i in range(_NCHUNK):
            b = i % 2
            nb = (i + 1) % 2
            if i + 1 < _NCHUNK:
                if i >= 1:
                    w_desc(i - 1, nb).wait()
                g_desc(i + 1, nb).start()
            g_desc(i, b).wait()
            w_desc(i, b).start()
        w_desc(_NCHUNK - 2, (_NCHUNK - 2) % 2).wait()
        w_desc(_NCHUNK - 1, (_NCHUNK - 1) % 2).wait()

    return k(idx_flat, table)


def kernel(inputs, table):
    v = inputs.astype(jnp.int32)
    vr = (
        (v & ~jnp.int32(511))
        + ((v & 127) << 2)
        + ((v >> 7) & 3)
    )
    idx_perm = (
        vr.reshape(_NW, 128, _MAX_LEN // 4, 4)
        .transpose(0, 2, 1, 3)
        .reshape(-1)
    )
    tlin = _relayout_tc(table.T)
    out = _embedding_sc(idx_perm, tlin.reshape(_VPAD, _EMBED))
    fmt = _format_tc(out.reshape(_B * _EMBED // 128, 128))
    return jnp.transpose(fmt, (2, 0, 1))
